# baseline (device time: 56983 ns/iter reference)
import functools

import jax
import jax.numpy as jnp
from jax import lax
from jax.experimental import pallas as pl
from jax.experimental.pallas import tpu as pltpu

N_DEV = 4
N_TOK = 1024
D_IN = 512
D_OUT = 1024
N_EXP = 16
EXP_PER_DEV = 4
ROWS_PER_DEV = N_TOK // N_DEV


def kernel(x, router_W, route_idx, expert_W):
    def body(x_ref, rw_ref, idx_ref, ew_ref, out_ref,
             acc_ref, comm_ref, send_sems, recv_sems):
        my = lax.axis_index("i")
        left = (my + N_DEV - 1) % N_DEV
        right = (my + 1) % N_DEV

        barrier_sem = pltpu.get_barrier_semaphore()
        for nbr in (left, right):
            pl.semaphore_signal(
                barrier_sem, inc=1,
                device_id=(nbr,), device_id_type=pl.DeviceIdType.MESH,
            )
        pl.semaphore_wait(barrier_sem, 2)

        xf = x_ref[:, :]
        scores = jnp.dot(xf, rw_ref[:, :], preferred_element_type=jnp.float32)
        p = jnp.exp(scores - jnp.max(scores, axis=1, keepdims=True))
        idx = idx_ref[:, :]
        eids = lax.broadcasted_iota(jnp.int32, (N_TOK, N_EXP), 1)
        sel = jnp.logical_or(idx[:, 0:1] == eids, idx[:, 1:2] == eids)
        psel = jnp.where(sel, p, 0.0)
        w = psel / jnp.sum(psel, axis=1, keepdims=True)

        cols = []
        for e in range(EXP_PER_DEV):
            gid = my * EXP_PER_DEV + e
            we = jnp.sum(jnp.where(eids == gid, w, 0.0), axis=1, keepdims=True)
            cols.append((xf * we).astype(jnp.bfloat16))
        xbig = jnp.concatenate(cols, axis=1)
        wbig = ew_ref[...].reshape(EXP_PER_DEV * D_IN, D_OUT).astype(jnp.bfloat16)
        acc_ref[:, :] = jnp.dot(xbig, wbig, preferred_element_type=jnp.float32)

        for s in range(N_DEV - 1):
            c_send = (my + 3 - s) % N_DEV
            c_recv = (my + 2 - s) % N_DEV
            rdma = pltpu.make_async_remote_copy(
                src_ref=acc_ref.at[pl.ds(c_send * ROWS_PER_DEV, ROWS_PER_DEV)],
                dst_ref=comm_ref.at[s],
                send_sem=send_sems.at[s],
                recv_sem=recv_sems.at[s],
                device_id=(right,),
                device_id_type=pl.DeviceIdType.MESH,
            )
            rdma.start()
            rdma.wait()
            r0 = c_recv * ROWS_PER_DEV
            acc_ref[pl.ds(r0, ROWS_PER_DEV)] = (
                acc_ref[pl.ds(r0, ROWS_PER_DEV)] + comm_ref[s]
            )

        out_ref[:, :] = acc_ref[pl.ds(my * ROWS_PER_DEV, ROWS_PER_DEV)]

        @functools.partial(pl.run_scoped, sem=pltpu.SemaphoreType.REGULAR)
        def _(sem):
            for nbr in (left, right):
                pl.semaphore_signal(
                    sem, inc=1,
                    device_id=(nbr,), device_id_type=pl.DeviceIdType.MESH,
                )
            pl.semaphore_wait(sem, 2)

    return pl.pallas_call(
        body,
        out_shape=jax.ShapeDtypeStruct((ROWS_PER_DEV, D_OUT), jnp.float32),
        in_specs=[pl.BlockSpec(memory_space=pltpu.VMEM)] * 4,
        out_specs=pl.BlockSpec(memory_space=pltpu.VMEM),
        scratch_shapes=[
            pltpu.VMEM((N_TOK, D_OUT), jnp.float32),
            pltpu.VMEM((N_DEV - 1, ROWS_PER_DEV, D_OUT), jnp.float32),
            pltpu.SemaphoreType.DMA((N_DEV - 1,)),
            pltpu.SemaphoreType.DMA((N_DEV - 1,)),
        ],
        compiler_params=pltpu.CompilerParams(collective_id=0),
    )(x, router_W, route_idx, expert_W)


# device time: 29954 ns/iter; 1.9024x vs baseline; 1.9024x over previous
import functools

import jax
import jax.numpy as jnp
from jax import lax
from jax.experimental import pallas as pl
from jax.experimental.pallas import tpu as pltpu

N_DEV = 4
N_TOK = 1024
D_IN = 512
D_OUT = 1024
N_EXP = 16
EXP_PER_DEV = 4
ROWS = N_TOK // N_DEV


def kernel(x, router_W, route_idx, expert_W):
    def body(x_ref, rw_ref, idx_ref, ew_ref, out_ref,
             parts_ref, rbuf_ref, w_ref, send_sems, recv_sems):
        my = lax.axis_index("i")
        peers = [(my + o) % N_DEV for o in (1, 2, 3)]

        barrier_sem = pltpu.get_barrier_semaphore()
        for p in peers:
            pl.semaphore_signal(
                barrier_sem, inc=1,
                device_id=(p,), device_id_type=pl.DeviceIdType.MESH,
            )
        pl.semaphore_wait(barrier_sem, N_DEV - 1)

        xf = x_ref[:, :]
        scores = jnp.dot(xf, rw_ref[:, :], preferred_element_type=jnp.float32)
        p = jnp.exp(scores - jnp.max(scores, axis=1, keepdims=True))
        idx = idx_ref[:, :]
        eids = lax.broadcasted_iota(jnp.int32, (N_TOK, N_EXP), 1)
        sel = jnp.logical_or(idx[:, 0:1] == eids, idx[:, 1:2] == eids)
        psel = jnp.where(sel, p, 0.0)
        w_ref[:, :] = psel / jnp.sum(psel, axis=1, keepdims=True)

        wbig = ew_ref[...].reshape(EXP_PER_DEV * D_IN, D_OUT).astype(jnp.bfloat16)

        def compute_chunk(c):
            r0 = c * ROWS
            xc = x_ref[pl.ds(r0, ROWS)]
            wc = w_ref[pl.ds(r0, ROWS)]
            eid_c = lax.broadcasted_iota(jnp.int32, (ROWS, N_EXP), 1)
            cols = []
            for e in range(EXP_PER_DEV):
                gid = my * EXP_PER_DEV + e
                we = jnp.sum(jnp.where(eid_c == gid, wc, 0.0),
                             axis=1, keepdims=True)
                cols.append((xc * we).astype(jnp.bfloat16))
            xbig = jnp.concatenate(cols, axis=1)
            return jnp.dot(xbig, wbig,
                           preferred_element_type=jnp.float32).astype(jnp.bfloat16)

        rdmas = []
        for o in (2, 1, 3):
            dest = (my + o) % N_DEV
            slot = 3 - o
            parts_ref[pl.ds(dest * ROWS, ROWS)] = compute_chunk(dest)
            rdma = pltpu.make_async_remote_copy(
                src_ref=parts_ref.at[pl.ds(dest * ROWS, ROWS)],
                dst_ref=rbuf_ref.at[slot],
                send_sem=send_sems.at[slot],
                recv_sem=recv_sems.at[slot],
                device_id=(dest,),
                device_id_type=pl.DeviceIdType.MESH,
            )
            rdma.start()
            rdmas.append(rdma)

        own = compute_chunk(my).astype(jnp.float32)

        for rdma in rdmas:
            rdma.wait()
        out_ref[:, :] = (
            own
            + rbuf_ref[0].astype(jnp.float32)
            + rbuf_ref[1].astype(jnp.float32)
            + rbuf_ref[2].astype(jnp.float32)
        )

        @functools.partial(pl.run_scoped, sem=pltpu.SemaphoreType.REGULAR)
        def _(sem):
            for p in peers:
                pl.semaphore_signal(
                    sem, inc=1,
                    device_id=(p,), device_id_type=pl.DeviceIdType.MESH,
                )
            pl.semaphore_wait(sem, N_DEV - 1)

    return pl.pallas_call(
        body,
        out_shape=jax.ShapeDtypeStruct((ROWS, D_OUT), jnp.float32),
        in_specs=[pl.BlockSpec(memory_space=pltpu.VMEM)] * 4,
        out_specs=pl.BlockSpec(memory_space=pltpu.VMEM),
        scratch_shapes=[
            pltpu.VMEM((N_TOK, D_OUT), jnp.bfloat16),
            pltpu.VMEM((N_DEV - 1, ROWS, D_OUT), jnp.bfloat16),
            pltpu.VMEM((N_TOK, N_EXP), jnp.float32),
            pltpu.SemaphoreType.DMA((N_DEV - 1,)),
            pltpu.SemaphoreType.DMA((N_DEV - 1,)),
        ],
        compiler_params=pltpu.CompilerParams(collective_id=0),
    )(x, router_W, route_idx, expert_W)
